# baseline (speedup 1.0000x reference)
import functools

import jax
import jax.numpy as jnp
from jax import lax
from jax.experimental import pallas as pl
from jax.experimental.pallas import tpu as pltpu
from jax.experimental.pallas import tpu_sc as plsc

E = 16
K = 2
D = 2048
F = 1024
T = 4096
A = T * K
BLK = 256
_BSH = BLK.bit_length() - 1
NBLK = A // BLK + E
P = NBLK * BLK
NC, NS, L = 2, 16, 16
NW = NC * NS

_SC_PARAMS = pltpu.CompilerParams(needs_layout_passes=False)


def _mesh():
    return plsc.VectorSubcoreMesh(core_axis_name="c", subcore_axis_name="s",
                                  num_cores=NC, num_subcores=NS)


_RT = 512


def _router_body(x_ref, gw_ref, w1_ref, w2_ref, i1_ref, i2_ref):
    x = x_ref[...]
    gw = gw_ref[...]
    logits = lax.dot_general(x, gw, (((1,), (1,)), ((), ())),
                             preferred_element_type=jnp.float32)
    m = jnp.max(logits, axis=-1, keepdims=True)
    ex = jnp.exp(logits - m)
    probs = ex / jnp.sum(ex, axis=-1, keepdims=True)
    i1 = jnp.argmax(probs, axis=-1).astype(jnp.int32)
    w1 = jnp.max(probs, axis=-1)
    cols = lax.broadcasted_iota(jnp.int32, probs.shape, 1)
    probs2 = jnp.where(cols == i1[:, None], -jnp.inf, probs)
    i2 = jnp.argmax(probs2, axis=-1).astype(jnp.int32)
    w2 = jnp.max(probs2, axis=-1)
    w1_ref[...] = w1
    w2_ref[...] = w2
    i1_ref[...] = i1
    i2_ref[...] = i2


def _router(x, gate_w):
    return pl.pallas_call(
        _router_body,
        grid=(T // _RT,),
        in_specs=[
            pl.BlockSpec((_RT, D), lambda i: (i, 0)),
            pl.BlockSpec((E, D), lambda i: (0, 0)),
        ],
        out_specs=[
            pl.BlockSpec((_RT,), lambda i: (i,)),
            pl.BlockSpec((_RT,), lambda i: (i,)),
            pl.BlockSpec((_RT,), lambda i: (i,)),
            pl.BlockSpec((_RT,), lambda i: (i,)),
        ],
        out_shape=[
            jax.ShapeDtypeStruct((T,), jnp.float32),
            jax.ShapeDtypeStruct((T,), jnp.float32),
            jax.ShapeDtypeStruct((T,), jnp.int32),
            jax.ShapeDtypeStruct((T,), jnp.int32),
        ],
    )(x, gate_w)


def _dispatch_body(i1_hbm, i2_hbm,
                   rowtok_hbm, dest_hbm, be_hbm, bv_hbm, xb_hbm,
                   idx_v, cnt2d, base2d, off_v, rowtok_v, dest_v,
                   be_v, bv_v, xb_v):
    cid = lax.axis_index("c")
    sid = lax.axis_index("s")

    @pl.when(jnp.logical_and(cid == 0, sid == 0))
    def _():
        lane = lax.iota(jnp.int32, L)
        zeros = jnp.zeros((L,), jnp.int32)
        lane_e = lane * E

        pltpu.sync_copy(i1_hbm, idx_v.at[pl.ds(0, T)])
        pltpu.sync_copy(i2_hbm, idx_v.at[pl.ds(T, T)])

        for c in range(L):
            cnt2d[pl.ds(c * E, E)] = zeros

        def p1(i, _):
            e = idx_v[pl.ds(i * L, L)]
            cur = plsc.load_gather(cnt2d, [lane_e + e])
            plsc.store_scatter(cnt2d, [lane_e + e], cur + 1)
            return 0
        lax.fori_loop(0, A // L, p1, 0)

        tot = cnt2d[pl.ds(0, E)]
        for c in range(1, L):
            tot = tot + cnt2d[pl.ds(c * E, E)]

        padded = ((tot + (BLK - 1)) >> _BSH) << _BSH
        nblk = (tot + (BLK - 1)) >> _BSH
        off = plsc.cumsum(padded) - padded
        blkoff = plsc.cumsum(nblk) - nblk
        total_blocks = jnp.sum(nblk)
        off_v[...] = blkoff

        for j in range(NBLK // L):
            b = lane + j * L
            acc = jnp.zeros((L,), jnp.int32)
            for e in range(E):
                oe = plsc.load_gather(off_v, [jnp.full((L,), e, jnp.int32)])
                acc = acc + (b >= oe).astype(jnp.int32)
            be_v[pl.ds(j * L, L)] = acc - 1
            bv_v[pl.ds(j * L, L)] = (b < total_blocks).astype(jnp.int32)
            xb_v[pl.ds(j * L, L)] = jnp.minimum(b, total_blocks - 1)

        acc_v = off
        for c in range(L):
            base2d[pl.ds(c * E, E)] = acc_v
            acc_v = acc_v + cnt2d[pl.ds(c * E, E)]

        def ms(j, _):
            rowtok_v[pl.ds(j * L, L)] = (j * L + lane) & (T - 1)
            return 0
        lax.fori_loop(0, P // L, ms, 0)

        def p2(i, _):
            e = idx_v[pl.ds(i * L, L)]
            cur = plsc.load_gather(base2d, [lane_e + e])
            plsc.store_scatter(base2d, [lane_e + e], cur + 1)
            a = i * L + lane
            tok = a & (T - 1)
            kflag = a >> 12
            plsc.store_scatter(dest_v, [(tok << 1) | kflag], cur)
            plsc.store_scatter(rowtok_v, [cur], tok)
            return 0
        lax.fori_loop(0, A // L, p2, 0)

        pltpu.sync_copy(rowtok_v, rowtok_hbm)
        pltpu.sync_copy(dest_v, dest_hbm)
        pltpu.sync_copy(be_v, be_hbm)
        pltpu.sync_copy(bv_v, bv_hbm)
        pltpu.sync_copy(xb_v, xb_hbm)


def _dispatch():
  return pl.kernel(
    _dispatch_body, mesh=_mesh(), compiler_params=_SC_PARAMS,
    out_type=[
        jax.ShapeDtypeStruct((P,), jnp.int32),
        jax.ShapeDtypeStruct((A,), jnp.int32),
        jax.ShapeDtypeStruct((NBLK,), jnp.int32),
        jax.ShapeDtypeStruct((NBLK,), jnp.int32),
        jax.ShapeDtypeStruct((NBLK,), jnp.int32),
    ],
    scratch_types=[
        pltpu.VMEM((A,), jnp.int32),
        pltpu.VMEM((L * E,), jnp.int32),
        pltpu.VMEM((L * E,), jnp.int32),
        pltpu.VMEM((E,), jnp.int32),
        pltpu.VMEM((P,), jnp.int32),
        pltpu.VMEM((A,), jnp.int32),
        pltpu.VMEM((NBLK,), jnp.int32),
        pltpu.VMEM((NBLK,), jnp.int32),
        pltpu.VMEM((NBLK,), jnp.int32),
    ],
)


_GC = 24
_GN = P // NW // _GC
_GB = 2


def _gather_body(x_hbm, rowtok_hbm, xg_hbm, idx_v, *rest):
    bufs = rest[:_GB]
    gsems = rest[_GB:2 * _GB]
    ssems = rest[2 * _GB:3 * _GB]
    wid = lax.axis_index("c") * NS + lax.axis_index("s")
    base = wid * (P // NW)
    pltpu.sync_copy(rowtok_hbm.at[pl.ds(base, P // NW)], idx_v)

    def gcpy(j, b):
        return pltpu.make_async_copy(
            x_hbm.at[idx_v.at[pl.ds(j * _GC, _GC)]], bufs[b], gsems[b])

    def scpy(j, b):
        return pltpu.make_async_copy(
            bufs[b], xg_hbm.at[pl.ds(base + j * _GC, _GC)], ssems[b])

    for b in range(_GB):
        gcpy(b, b).start()

    @pl.loop(0, _GN, step=_GB)
    def _(g):
        for b in range(_GB):
            j = g + b
            gcpy(j, b).wait()
            scpy(j, b).start()
            scpy(j, b).wait()

            @pl.when(j + _GB < _GN)
            def _():
                gcpy(j + _GB, b).start()


def _gather():
  return pl.kernel(
    _gather_body, mesh=_mesh(), compiler_params=_SC_PARAMS,
    out_type=jax.ShapeDtypeStruct((P, D), jnp.float32),
    scratch_types=(
        [pltpu.VMEM((P // NW,), jnp.int32)]
        + [pltpu.VMEM((_GC, D), jnp.float32)] * _GB
        + [pltpu.SemaphoreType.DMA] * (2 * _GB)
    ),
)


def _expert_body(be_ref, bv_ref, xb_ref, xg_ref, w1_ref, w3_ref, w2_ref,
                 yg_ref):
    i = pl.program_id(0)

    @pl.when(bv_ref[i] == 1)
    def _():
        x = xg_ref[...].astype(jnp.float32)
        h1 = lax.dot_general(x, w1_ref[0], (((1,), (1,)), ((), ())),
                             preferred_element_type=jnp.float32)
        h3 = lax.dot_general(x, w3_ref[0], (((1,), (1,)), ((), ())),
                             preferred_element_type=jnp.float32)
        h = h1 * lax.logistic(h1) * h3
        yg_ref[...] = lax.dot_general(h, w2_ref[0], (((1,), (1,)), ((), ())),
                                      preferred_element_type=jnp.float32)


def _experts(xg, w1, w3, w2, be, bv, xb):
    grid_spec = pltpu.PrefetchScalarGridSpec(
        num_scalar_prefetch=3,
        grid=(NBLK,),
        in_specs=[
            pl.BlockSpec((BLK, D), lambda i, be, bv, xb: (xb[i], 0)),
            pl.BlockSpec((1, F, D), lambda i, be, bv, xb: (be[i], 0, 0)),
            pl.BlockSpec((1, F, D), lambda i, be, bv, xb: (be[i], 0, 0)),
            pl.BlockSpec((1, D, F), lambda i, be, bv, xb: (be[i], 0, 0)),
        ],
        out_specs=pl.BlockSpec((BLK, D), lambda i, be, bv, xb: (xb[i], 0)),
    )
    return pl.pallas_call(
        _expert_body,
        grid_spec=grid_spec,
        out_shape=jax.ShapeDtypeStruct((P, D), jnp.float32),
    )(be, bv, xb, xg, w1, w3, w2)


_CT = 8
_CN = T // NW // _CT
_CB = 3


def _combine_body(yg_hbm, dest_hbm, wa_hbm, wb_hbm, o_hbm,
                  dv, wv0, wv1, buf0, buf1, buf2, obuf,
                  g0, g1, g2, ssem):
    bufs, gsems = (buf0, buf1, buf2), (g0, g1, g2)
    wid = lax.axis_index("c") * NS + lax.axis_index("s")
    tt = T // NW
    base = wid * tt
    pltpu.sync_copy(dest_hbm.at[pl.ds(2 * base, 2 * tt)], dv)
    pltpu.sync_copy(wa_hbm.at[pl.ds(base, tt)], wv0)
    pltpu.sync_copy(wb_hbm.at[pl.ds(base, tt)], wv1)

    def gcpy(j, b):
        return pltpu.make_async_copy(
            yg_hbm.at[dv.at[pl.ds(j * 2 * _CT, 2 * _CT)]], bufs[b], gsems[b])

    def scpy(j):
        return pltpu.make_async_copy(
            obuf, o_hbm.at[pl.ds(base + j * _CT, _CT)], ssem)

    for b in range(_CB):
        gcpy(b, b).start()

    for j in range(_CN):
        b = j % _CB
        gcpy(j, b).wait()
        if j >= 1:
            scpy(j - 1).wait()
        for u in range(_CT):
            iu = jnp.full((L,), j * _CT + u, jnp.int32)
            wa = plsc.load_gather(wv0, [iu])
            wb = plsc.load_gather(wv1, [iu])

            @plsc.parallel_loop(0, D // L, unroll=4)
            def _(v, u=u, wa=wa, wb=wb, b=b):
                av = bufs[b][2 * u, pl.ds(v * L, L)]
                bv = bufs[b][2 * u + 1, pl.ds(v * L, L)]
                obuf[u, pl.ds(v * L, L)] = wa * av + wb * bv
        scpy(j).start()
        if j + _CB < _CN:
            gcpy(j + _CB, b).start()
    scpy(_CN - 1).wait()


def _combine():
  return pl.kernel(
    _combine_body, mesh=_mesh(), compiler_params=_SC_PARAMS,
    out_type=jax.ShapeDtypeStruct((T, D), jnp.float32),
    scratch_types=[
        pltpu.VMEM((2 * T // NW,), jnp.int32),
        pltpu.VMEM((T // NW,), jnp.float32),
        pltpu.VMEM((T // NW,), jnp.float32),
        pltpu.VMEM((2 * _CT, D), jnp.float32),
        pltpu.VMEM((2 * _CT, D), jnp.float32),
        pltpu.VMEM((2 * _CT, D), jnp.float32),
        pltpu.VMEM((_CT, D), jnp.float32),
        pltpu.SemaphoreType.DMA,
        pltpu.SemaphoreType.DMA,
        pltpu.SemaphoreType.DMA,
        pltpu.SemaphoreType.DMA,
    ],
)


def kernel(hidden_states, gate_w, w1, w3, w2):
    orig_shape = hidden_states.shape
    x = hidden_states.reshape(-1, D)
    wa, wb, i1, i2 = _router(x, gate_w)
    rowtok, dest, be, bv, xb = _dispatch()(i1, i2)
    xg = _gather()(x, rowtok)
    yg = _experts(xg, w1, w3, w2, be, bv, xb)
    out = _combine()(yg, dest, wa, wb)
    return out.reshape(orig_shape)


# speedup vs baseline: 1.0026x; 1.0026x over previous
import functools

import jax
import jax.numpy as jnp
from jax import lax
from jax.experimental import pallas as pl
from jax.experimental.pallas import tpu as pltpu
from jax.experimental.pallas import tpu_sc as plsc

E = 16
K = 2
D = 2048
F = 1024
T = 4096
A = T * K
BLK = 256
_BSH = BLK.bit_length() - 1
NBLK = A // BLK + E
P = NBLK * BLK
NC, NS, L = 2, 16, 16
NW = NC * NS

_SC_PARAMS = pltpu.CompilerParams(needs_layout_passes=False)


def _mesh():
    return plsc.VectorSubcoreMesh(core_axis_name="c", subcore_axis_name="s",
                                  num_cores=NC, num_subcores=NS)


_RT = 1024


def _router_body(x_ref, gw_ref, w1_ref, w2_ref, i1_ref, i2_ref):
    x = x_ref[...]
    gw = gw_ref[...]
    logits = lax.dot_general(x, gw, (((1,), (1,)), ((), ())),
                             preferred_element_type=jnp.float32)
    m = jnp.max(logits, axis=-1, keepdims=True)
    ex = jnp.exp(logits - m)
    probs = ex / jnp.sum(ex, axis=-1, keepdims=True)
    i1 = jnp.argmax(probs, axis=-1).astype(jnp.int32)
    w1 = jnp.max(probs, axis=-1)
    cols = lax.broadcasted_iota(jnp.int32, probs.shape, 1)
    probs2 = jnp.where(cols == i1[:, None], -jnp.inf, probs)
    i2 = jnp.argmax(probs2, axis=-1).astype(jnp.int32)
    w2 = jnp.max(probs2, axis=-1)
    w1_ref[...] = w1
    w2_ref[...] = w2
    i1_ref[...] = i1
    i2_ref[...] = i2


def _router(x, gate_w):
    return pl.pallas_call(
        _router_body,
        grid=(T // _RT,),
        in_specs=[
            pl.BlockSpec((_RT, D), lambda i: (i, 0)),
            pl.BlockSpec((E, D), lambda i: (0, 0)),
        ],
        out_specs=[
            pl.BlockSpec((_RT,), lambda i: (i,)),
            pl.BlockSpec((_RT,), lambda i: (i,)),
            pl.BlockSpec((_RT,), lambda i: (i,)),
            pl.BlockSpec((_RT,), lambda i: (i,)),
        ],
        out_shape=[
            jax.ShapeDtypeStruct((T,), jnp.float32),
            jax.ShapeDtypeStruct((T,), jnp.float32),
            jax.ShapeDtypeStruct((T,), jnp.int32),
            jax.ShapeDtypeStruct((T,), jnp.int32),
        ],
    )(x, gate_w)


def _dispatch_body(i1_hbm, i2_hbm,
                   rowtok_hbm, dest_hbm, be_hbm, bv_hbm, xb_hbm,
                   idx_v, cnt2d, base2d, off_v, rowtok_v, dest_v,
                   be_v, bv_v, xb_v):
    cid = lax.axis_index("c")
    sid = lax.axis_index("s")

    @pl.when(jnp.logical_and(cid == 0, sid == 0))
    def _():
        lane = lax.iota(jnp.int32, L)
        zeros = jnp.zeros((L,), jnp.int32)
        lane_e = lane * E

        pltpu.sync_copy(i1_hbm, idx_v.at[pl.ds(0, T)])
        pltpu.sync_copy(i2_hbm, idx_v.at[pl.ds(T, T)])

        for c in range(L):
            cnt2d[pl.ds(c * E, E)] = zeros

        def p1(i, _):
            e = idx_v[pl.ds(i * L, L)]
            cur = plsc.load_gather(cnt2d, [lane_e + e])
            plsc.store_scatter(cnt2d, [lane_e + e], cur + 1)
            return 0
        lax.fori_loop(0, A // L, p1, 0)

        tot = cnt2d[pl.ds(0, E)]
        for c in range(1, L):
            tot = tot + cnt2d[pl.ds(c * E, E)]

        padded = ((tot + (BLK - 1)) >> _BSH) << _BSH
        nblk = (tot + (BLK - 1)) >> _BSH
        off = plsc.cumsum(padded) - padded
        blkoff = plsc.cumsum(nblk) - nblk
        total_blocks = jnp.sum(nblk)
        off_v[...] = blkoff

        for j in range(NBLK // L):
            b = lane + j * L
            acc = jnp.zeros((L,), jnp.int32)
            for e in range(E):
                oe = plsc.load_gather(off_v, [jnp.full((L,), e, jnp.int32)])
                acc = acc + (b >= oe).astype(jnp.int32)
            be_v[pl.ds(j * L, L)] = acc - 1
            bv_v[pl.ds(j * L, L)] = (b < total_blocks).astype(jnp.int32)
            xb_v[pl.ds(j * L, L)] = jnp.minimum(b, total_blocks - 1)

        acc_v = off
        for c in range(L):
            base2d[pl.ds(c * E, E)] = acc_v
            acc_v = acc_v + cnt2d[pl.ds(c * E, E)]

        def ms(j, _):
            rowtok_v[pl.ds(j * L, L)] = (j * L + lane) & (T - 1)
            return 0
        lax.fori_loop(0, P // L, ms, 0)

        def p2(i, _):
            e = idx_v[pl.ds(i * L, L)]
            cur = plsc.load_gather(base2d, [lane_e + e])
            plsc.store_scatter(base2d, [lane_e + e], cur + 1)
            a = i * L + lane
            tok = a & (T - 1)
            kflag = a >> 12
            plsc.store_scatter(dest_v, [(tok << 1) | kflag], cur)
            plsc.store_scatter(rowtok_v, [cur], tok)
            return 0
        lax.fori_loop(0, A // L, p2, 0)

        pltpu.sync_copy(rowtok_v, rowtok_hbm)
        pltpu.sync_copy(dest_v, dest_hbm)
        pltpu.sync_copy(be_v, be_hbm)
        pltpu.sync_copy(bv_v, bv_hbm)
        pltpu.sync_copy(xb_v, xb_hbm)


def _dispatch():
  return pl.kernel(
    _dispatch_body, mesh=_mesh(), compiler_params=_SC_PARAMS,
    out_type=[
        jax.ShapeDtypeStruct((P,), jnp.int32),
        jax.ShapeDtypeStruct((A,), jnp.int32),
        jax.ShapeDtypeStruct((NBLK,), jnp.int32),
        jax.ShapeDtypeStruct((NBLK,), jnp.int32),
        jax.ShapeDtypeStruct((NBLK,), jnp.int32),
    ],
    scratch_types=[
        pltpu.VMEM((A,), jnp.int32),
        pltpu.VMEM((L * E,), jnp.int32),
        pltpu.VMEM((L * E,), jnp.int32),
        pltpu.VMEM((E,), jnp.int32),
        pltpu.VMEM((P,), jnp.int32),
        pltpu.VMEM((A,), jnp.int32),
        pltpu.VMEM((NBLK,), jnp.int32),
        pltpu.VMEM((NBLK,), jnp.int32),
        pltpu.VMEM((NBLK,), jnp.int32),
    ],
)


_GC = 8
_GN = P // NW // _GC
_GB = 4


def _gather_body(x_hbm, rowtok_hbm, xg_hbm, idx_v, *rest):
    bufs = rest[:_GB]
    gsems = rest[_GB:2 * _GB]
    ssems = rest[2 * _GB:3 * _GB]
    wid = lax.axis_index("c") * NS + lax.axis_index("s")
    base = wid * (P // NW)
    pltpu.sync_copy(rowtok_hbm.at[pl.ds(base, P // NW)], idx_v)

    def gcpy(j, b):
        return pltpu.make_async_copy(
            x_hbm.at[idx_v.at[pl.ds(j * _GC, _GC)]], bufs[b], gsems[b])

    def scpy(j, b):
        return pltpu.make_async_copy(
            bufs[b], xg_hbm.at[pl.ds(base + j * _GC, _GC)], ssems[b])

    for b in range(_GB):
        gcpy(b, b).start()

    @pl.loop(0, _GN, step=_GB)
    def _(g):
        for b in range(_GB):
            j = g + b
            gcpy(j, b).wait()
            scpy(j, b).start()
            scpy(j, b).wait()

            @pl.when(j + _GB < _GN)
            def _():
                gcpy(j + _GB, b).start()


def _gather():
  return pl.kernel(
    _gather_body, mesh=_mesh(), compiler_params=_SC_PARAMS,
    out_type=jax.ShapeDtypeStruct((P, D), jnp.float32),
    scratch_types=(
        [pltpu.VMEM((P // NW,), jnp.int32)]
        + [pltpu.VMEM((_GC, D), jnp.float32)] * _GB
        + [pltpu.SemaphoreType.DMA] * (2 * _GB)
    ),
)


def _expert_body(be_ref, bv_ref, xb_ref, xg_ref, w1_ref, w3_ref, w2_ref,
                 yg_ref):
    i = pl.program_id(0)

    @pl.when(bv_ref[i] == 1)
    def _():
        x = xg_ref[...].astype(jnp.float32)
        h1 = lax.dot_general(x, w1_ref[0], (((1,), (1,)), ((), ())),
                             preferred_element_type=jnp.float32)
        h3 = lax.dot_general(x, w3_ref[0], (((1,), (1,)), ((), ())),
                             preferred_element_type=jnp.float32)
        h = h1 * lax.logistic(h1) * h3
        yg_ref[...] = lax.dot_general(h, w2_ref[0], (((1,), (1,)), ((), ())),
                                      preferred_element_type=jnp.float32)


def _experts(xg, w1, w3, w2, be, bv, xb):
    grid_spec = pltpu.PrefetchScalarGridSpec(
        num_scalar_prefetch=3,
        grid=(NBLK,),
        in_specs=[
            pl.BlockSpec((BLK, D), lambda i, be, bv, xb: (xb[i], 0)),
            pl.BlockSpec((1, F, D), lambda i, be, bv, xb: (be[i], 0, 0)),
            pl.BlockSpec((1, F, D), lambda i, be, bv, xb: (be[i], 0, 0)),
            pl.BlockSpec((1, D, F), lambda i, be, bv, xb: (be[i], 0, 0)),
        ],
        out_specs=pl.BlockSpec((BLK, D), lambda i, be, bv, xb: (xb[i], 0)),
    )
    return pl.pallas_call(
        _expert_body,
        grid_spec=grid_spec,
        out_shape=jax.ShapeDtypeStruct((P, D), jnp.float32),
    )(be, bv, xb, xg, w1, w3, w2)


_CT = 8
_CN = T // NW // _CT
_CB = 3


def _combine_body(yg_hbm, dest_hbm, wa_hbm, wb_hbm, o_hbm,
                  dv, wv0, wv1, buf0, buf1, buf2, obuf,
                  g0, g1, g2, ssem):
    bufs, gsems = (buf0, buf1, buf2), (g0, g1, g2)
    wid = lax.axis_index("c") * NS + lax.axis_index("s")
    tt = T // NW
    base = wid * tt
    pltpu.sync_copy(dest_hbm.at[pl.ds(2 * base, 2 * tt)], dv)
    pltpu.sync_copy(wa_hbm.at[pl.ds(base, tt)], wv0)
    pltpu.sync_copy(wb_hbm.at[pl.ds(base, tt)], wv1)

    def gcpy(j, b):
        return pltpu.make_async_copy(
            yg_hbm.at[dv.at[pl.ds(j * 2 * _CT, 2 * _CT)]], bufs[b], gsems[b])

    def scpy(j):
        return pltpu.make_async_copy(
            obuf, o_hbm.at[pl.ds(base + j * _CT, _CT)], ssem)

    for b in range(_CB):
        gcpy(b, b).start()

    for j in range(_CN):
        b = j % _CB
        gcpy(j, b).wait()
        if j >= 1:
            scpy(j - 1).wait()
        for u in range(_CT):
            iu = jnp.full((L,), j * _CT + u, jnp.int32)
            wa = plsc.load_gather(wv0, [iu])
            wb = plsc.load_gather(wv1, [iu])

            @plsc.parallel_loop(0, D // L, unroll=4)
            def _(v, u=u, wa=wa, wb=wb, b=b):
                av = bufs[b][2 * u, pl.ds(v * L, L)]
                bv = bufs[b][2 * u + 1, pl.ds(v * L, L)]
                obuf[u, pl.ds(v * L, L)] = wa * av + wb * bv
        scpy(j).start()
        if j + _CB < _CN:
            gcpy(j + _CB, b).start()
    scpy(_CN - 1).wait()


def _combine():
  return pl.kernel(
    _combine_body, mesh=_mesh(), compiler_params=_SC_PARAMS,
    out_type=jax.ShapeDtypeStruct((T, D), jnp.float32),
    scratch_types=[
        pltpu.VMEM((2 * T // NW,), jnp.int32),
        pltpu.VMEM((T // NW,), jnp.float32),
        pltpu.VMEM((T // NW,), jnp.float32),
        pltpu.VMEM((2 * _CT, D), jnp.float32),
        pltpu.VMEM((2 * _CT, D), jnp.float32),
        pltpu.VMEM((2 * _CT, D), jnp.float32),
        pltpu.VMEM((_CT, D), jnp.float32),
        pltpu.SemaphoreType.DMA,
        pltpu.SemaphoreType.DMA,
        pltpu.SemaphoreType.DMA,
        pltpu.SemaphoreType.DMA,
    ],
)


def kernel(hidden_states, gate_w, w1, w3, w2):
    orig_shape = hidden_states.shape
    x = hidden_states.reshape(-1, D)
    wa, wb, i1, i2 = _router(x, gate_w)
    rowtok, dest, be, bv, xb = _dispatch()(i1, i2)
    xg = _gather()(x, rowtok)
    yg = _experts(xg, w1, w3, w2, be, bv, xb)
    out = _combine()(yg, dest, wa, wb)
    return out.reshape(orig_shape)


# speedup vs baseline: 1.0040x; 1.0014x over previous
import functools

import jax
import jax.numpy as jnp
from jax import lax
from jax.experimental import pallas as pl
from jax.experimental.pallas import tpu as pltpu
from jax.experimental.pallas import tpu_sc as plsc

E = 16
K = 2
D = 2048
F = 1024
T = 4096
A = T * K
BLK = 256
_BSH = BLK.bit_length() - 1
NBLK = A // BLK + E
P = NBLK * BLK
NC, NS, L = 2, 16, 16
NW = NC * NS

_SC_PARAMS = pltpu.CompilerParams(needs_layout_passes=False)


def _mesh():
    return plsc.VectorSubcoreMesh(core_axis_name="c", subcore_axis_name="s",
                                  num_cores=NC, num_subcores=NS)


_RT = 512


def _router_body(x_ref, gw_ref, w1_ref, w2_ref, i1_ref, i2_ref):
    x = x_ref[...]
    gw = gw_ref[...]
    logits = lax.dot_general(x, gw, (((1,), (1,)), ((), ())),
                             preferred_element_type=jnp.float32)
    m = jnp.max(logits, axis=-1, keepdims=True)
    ex = jnp.exp(logits - m)
    probs = ex / jnp.sum(ex, axis=-1, keepdims=True)
    i1 = jnp.argmax(probs, axis=-1).astype(jnp.int32)
    w1 = jnp.max(probs, axis=-1)
    cols = lax.broadcasted_iota(jnp.int32, probs.shape, 1)
    probs2 = jnp.where(cols == i1[:, None], -jnp.inf, probs)
    i2 = jnp.argmax(probs2, axis=-1).astype(jnp.int32)
    w2 = jnp.max(probs2, axis=-1)
    w1_ref[...] = w1
    w2_ref[...] = w2
    i1_ref[...] = i1
    i2_ref[...] = i2


def _router(x, gate_w):
    return pl.pallas_call(
        _router_body,
        grid=(T // _RT,),
        in_specs=[
            pl.BlockSpec((_RT, D), lambda i: (i, 0)),
            pl.BlockSpec((E, D), lambda i: (0, 0)),
        ],
        out_specs=[
            pl.BlockSpec((_RT,), lambda i: (i,)),
            pl.BlockSpec((_RT,), lambda i: (i,)),
            pl.BlockSpec((_RT,), lambda i: (i,)),
            pl.BlockSpec((_RT,), lambda i: (i,)),
        ],
        out_shape=[
            jax.ShapeDtypeStruct((T,), jnp.float32),
            jax.ShapeDtypeStruct((T,), jnp.float32),
            jax.ShapeDtypeStruct((T,), jnp.int32),
            jax.ShapeDtypeStruct((T,), jnp.int32),
        ],
    )(x, gate_w)


def _dispatch_body(i1_hbm, i2_hbm,
                   rowtok_hbm, dest_hbm, be_hbm, bv_hbm, xb_hbm,
                   idx_v, cnt2d, base2d, off_v, rowtok_v, dest_v,
                   be_v, bv_v, xb_v):
    cid = lax.axis_index("c")
    sid = lax.axis_index("s")

    @pl.when(jnp.logical_and(cid == 0, sid == 0))
    def _():
        lane = lax.iota(jnp.int32, L)
        zeros = jnp.zeros((L,), jnp.int32)
        lane_e = lane * E

        pltpu.sync_copy(i1_hbm, idx_v.at[pl.ds(0, T)])
        pltpu.sync_copy(i2_hbm, idx_v.at[pl.ds(T, T)])

        for c in range(L):
            cnt2d[pl.ds(c * E, E)] = zeros

        def p1(i, _):
            e = idx_v[pl.ds(i * L, L)]
            cur = plsc.load_gather(cnt2d, [lane_e + e])
            plsc.store_scatter(cnt2d, [lane_e + e], cur + 1)
            return 0
        lax.fori_loop(0, A // L, p1, 0)

        tot = cnt2d[pl.ds(0, E)]
        for c in range(1, L):
            tot = tot + cnt2d[pl.ds(c * E, E)]

        padded = ((tot + (BLK - 1)) >> _BSH) << _BSH
        nblk = (tot + (BLK - 1)) >> _BSH
        off = plsc.cumsum(padded) - padded
        blkoff = plsc.cumsum(nblk) - nblk
        total_blocks = jnp.sum(nblk)
        off_v[...] = blkoff

        for j in range(NBLK // L):
            b = lane + j * L
            acc = jnp.zeros((L,), jnp.int32)
            for e in range(E):
                oe = plsc.load_gather(off_v, [jnp.full((L,), e, jnp.int32)])
                acc = acc + (b >= oe).astype(jnp.int32)
            be_v[pl.ds(j * L, L)] = acc - 1
            bv_v[pl.ds(j * L, L)] = (b < total_blocks).astype(jnp.int32)
            xb_v[pl.ds(j * L, L)] = jnp.minimum(b, total_blocks - 1)

        acc_v = off
        for c in range(L):
            base2d[pl.ds(c * E, E)] = acc_v
            acc_v = acc_v + cnt2d[pl.ds(c * E, E)]

        def ms(j, _):
            rowtok_v[pl.ds(j * L, L)] = (j * L + lane) & (T - 1)
            return 0
        lax.fori_loop(0, P // L, ms, 0)

        def p2(i, _):
            e = idx_v[pl.ds(i * L, L)]
            cur = plsc.load_gather(base2d, [lane_e + e])
            plsc.store_scatter(base2d, [lane_e + e], cur + 1)
            a = i * L + lane
            tok = a & (T - 1)
            kflag = a >> 12
            plsc.store_scatter(dest_v, [(tok << 1) | kflag], cur)
            plsc.store_scatter(rowtok_v, [cur], tok)
            return 0
        lax.fori_loop(0, A // L, p2, 0)

        pltpu.sync_copy(rowtok_v, rowtok_hbm)
        pltpu.sync_copy(dest_v, dest_hbm)
        pltpu.sync_copy(be_v, be_hbm)
        pltpu.sync_copy(bv_v, bv_hbm)
        pltpu.sync_copy(xb_v, xb_hbm)


def _dispatch():
  return pl.kernel(
    _dispatch_body, mesh=_mesh(), compiler_params=_SC_PARAMS,
    out_type=[
        jax.ShapeDtypeStruct((P,), jnp.int32),
        jax.ShapeDtypeStruct((A,), jnp.int32),
        jax.ShapeDtypeStruct((NBLK,), jnp.int32),
        jax.ShapeDtypeStruct((NBLK,), jnp.int32),
        jax.ShapeDtypeStruct((NBLK,), jnp.int32),
    ],
    scratch_types=[
        pltpu.VMEM((A,), jnp.int32),
        pltpu.VMEM((L * E,), jnp.int32),
        pltpu.VMEM((L * E,), jnp.int32),
        pltpu.VMEM((E,), jnp.int32),
        pltpu.VMEM((P,), jnp.int32),
        pltpu.VMEM((A,), jnp.int32),
        pltpu.VMEM((NBLK,), jnp.int32),
        pltpu.VMEM((NBLK,), jnp.int32),
        pltpu.VMEM((NBLK,), jnp.int32),
    ],
)


_GC = 8
_GN = P // NW // _GC
_GB = 4


def _gather_body(x_hbm, rowtok_hbm, xg_hbm, idx_v, *rest):
    bufs = rest[:_GB]
    gsems = rest[_GB:2 * _GB]
    ssems = rest[2 * _GB:3 * _GB]
    wid = lax.axis_index("c") * NS + lax.axis_index("s")
    base = wid * (P // NW)
    pltpu.sync_copy(rowtok_hbm.at[pl.ds(base, P // NW)], idx_v)

    def gcpy(j, b):
        return pltpu.make_async_copy(
            x_hbm.at[idx_v.at[pl.ds(j * _GC, _GC)]], bufs[b], gsems[b])

    def scpy(j, b):
        return pltpu.make_async_copy(
            bufs[b], xg_hbm.at[pl.ds(base + j * _GC, _GC)], ssems[b])

    for b in range(_GB):
        gcpy(b, b).start()

    @pl.loop(0, _GN, step=_GB)
    def _(g):
        for b in range(_GB):
            j = g + b
            gcpy(j, b).wait()
            scpy(j, b).start()
            scpy(j, b).wait()

            @pl.when(j + _GB < _GN)
            def _():
                gcpy(j + _GB, b).start()


def _gather():
  return pl.kernel(
    _gather_body, mesh=_mesh(), compiler_params=_SC_PARAMS,
    out_type=jax.ShapeDtypeStruct((P, D), jnp.float32),
    scratch_types=(
        [pltpu.VMEM((P // NW,), jnp.int32)]
        + [pltpu.VMEM((_GC, D), jnp.float32)] * _GB
        + [pltpu.SemaphoreType.DMA] * (2 * _GB)
    ),
)


def _expert_body(be_ref, bv_ref, xb_ref, xg_ref, w1_ref, w3_ref, w2_ref,
                 yg_ref):
    i = pl.program_id(0)

    @pl.when(bv_ref[i] == 1)
    def _():
        x = xg_ref[...].astype(jnp.float32)
        h1 = lax.dot_general(x, w1_ref[0], (((1,), (1,)), ((), ())),
                             preferred_element_type=jnp.float32)
        h3 = lax.dot_general(x, w3_ref[0], (((1,), (1,)), ((), ())),
                             preferred_element_type=jnp.float32)
        h = h1 * lax.logistic(h1) * h3
        yg_ref[...] = lax.dot_general(h, w2_ref[0], (((1,), (1,)), ((), ())),
                                      preferred_element_type=jnp.float32)


def _experts(xg, w1, w3, w2, be, bv, xb):
    grid_spec = pltpu.PrefetchScalarGridSpec(
        num_scalar_prefetch=3,
        grid=(NBLK,),
        in_specs=[
            pl.BlockSpec((BLK, D), lambda i, be, bv, xb: (xb[i], 0)),
            pl.BlockSpec((1, F, D), lambda i, be, bv, xb: (be[i], 0, 0)),
            pl.BlockSpec((1, F, D), lambda i, be, bv, xb: (be[i], 0, 0)),
            pl.BlockSpec((1, D, F), lambda i, be, bv, xb: (be[i], 0, 0)),
        ],
        out_specs=pl.BlockSpec((BLK, D), lambda i, be, bv, xb: (xb[i], 0)),
    )
    return pl.pallas_call(
        _expert_body,
        grid_spec=grid_spec,
        out_shape=jax.ShapeDtypeStruct((P, D), jnp.float32),
    )(be, bv, xb, xg, w1, w3, w2)


_CT = 8
_CN = T // NW // _CT
_CB = 3


def _combine_body(yg_hbm, dest_hbm, wa_hbm, wb_hbm, o_hbm,
                  dv, wv0, wv1, buf0, buf1, buf2, obuf,
                  g0, g1, g2, ssem):
    bufs, gsems = (buf0, buf1, buf2), (g0, g1, g2)
    wid = lax.axis_index("c") * NS + lax.axis_index("s")
    tt = T // NW
    base = wid * tt
    pltpu.sync_copy(dest_hbm.at[pl.ds(2 * base, 2 * tt)], dv)
    pltpu.sync_copy(wa_hbm.at[pl.ds(base, tt)], wv0)
    pltpu.sync_copy(wb_hbm.at[pl.ds(base, tt)], wv1)

    def gcpy(j, b):
        return pltpu.make_async_copy(
            yg_hbm.at[dv.at[pl.ds(j * 2 * _CT, 2 * _CT)]], bufs[b], gsems[b])

    def scpy(j):
        return pltpu.make_async_copy(
            obuf, o_hbm.at[pl.ds(base + j * _CT, _CT)], ssem)

    for b in range(_CB):
        gcpy(b, b).start()

    for j in range(_CN):
        b = j % _CB
        gcpy(j, b).wait()
        if j >= 1:
            scpy(j - 1).wait()
        for u in range(_CT):
            iu = jnp.full((L,), j * _CT + u, jnp.int32)
            wa = plsc.load_gather(wv0, [iu])
            wb = plsc.load_gather(wv1, [iu])

            @plsc.parallel_loop(0, D // L, unroll=4)
            def _(v, u=u, wa=wa, wb=wb, b=b):
                av = bufs[b][2 * u, pl.ds(v * L, L)]
                bv = bufs[b][2 * u + 1, pl.ds(v * L, L)]
                obuf[u, pl.ds(v * L, L)] = wa * av + wb * bv
        scpy(j).start()
        if j + _CB < _CN:
            gcpy(j + _CB, b).start()
    scpy(_CN - 1).wait()


def _combine():
  return pl.kernel(
    _combine_body, mesh=_mesh(), compiler_params=_SC_PARAMS,
    out_type=jax.ShapeDtypeStruct((T, D), jnp.float32),
    scratch_types=[
        pltpu.VMEM((2 * T // NW,), jnp.int32),
        pltpu.VMEM((T // NW,), jnp.float32),
        pltpu.VMEM((T // NW,), jnp.float32),
        pltpu.VMEM((2 * _CT, D), jnp.float32),
        pltpu.VMEM((2 * _CT, D), jnp.float32),
        pltpu.VMEM((2 * _CT, D), jnp.float32),
        pltpu.VMEM((_CT, D), jnp.float32),
        pltpu.SemaphoreType.DMA,
        pltpu.SemaphoreType.DMA,
        pltpu.SemaphoreType.DMA,
        pltpu.SemaphoreType.DMA,
    ],
)


def kernel(hidden_states, gate_w, w1, w3, w2):
    orig_shape = hidden_states.shape
    x = hidden_states.reshape(-1, D)
    wa, wb, i1, i2 = _router(x, gate_w)
    rowtok, dest, be, bv, xb = _dispatch()(i1, i2)
    xg = _gather()(x, rowtok)
    yg = _experts(xg, w1, w3, w2, be, bv, xb)
    out = _combine()(yg, dest, wa, wb)
    return out.reshape(orig_shape)


# speedup vs baseline: 1.0077x; 1.0037x over previous
import functools

import jax
import jax.numpy as jnp
from jax import lax
from jax.experimental import pallas as pl
from jax.experimental.pallas import tpu as pltpu
from jax.experimental.pallas import tpu_sc as plsc

E = 16
K = 2
D = 2048
F = 1024
T = 4096
A = T * K
BLK = 256
_BSH = BLK.bit_length() - 1
NBLK = A // BLK + E
P = NBLK * BLK
NC, NS, L = 2, 16, 16
NW = NC * NS

_SC_PARAMS = pltpu.CompilerParams(needs_layout_passes=False)


def _mesh():
    return plsc.VectorSubcoreMesh(core_axis_name="c", subcore_axis_name="s",
                                  num_cores=NC, num_subcores=NS)


_RT = 512


def _router_body(x_ref, gw_ref, w1_ref, w2_ref, i1_ref, i2_ref):
    x = x_ref[...]
    gw = gw_ref[...]
    logits = lax.dot_general(x, gw, (((1,), (1,)), ((), ())),
                             preferred_element_type=jnp.float32)
    m = jnp.max(logits, axis=-1, keepdims=True)
    ex = jnp.exp(logits - m)
    probs = ex / jnp.sum(ex, axis=-1, keepdims=True)
    i1 = jnp.argmax(probs, axis=-1).astype(jnp.int32)
    w1 = jnp.max(probs, axis=-1)
    cols = lax.broadcasted_iota(jnp.int32, probs.shape, 1)
    probs2 = jnp.where(cols == i1[:, None], -jnp.inf, probs)
    i2 = jnp.argmax(probs2, axis=-1).astype(jnp.int32)
    w2 = jnp.max(probs2, axis=-1)
    w1_ref[...] = w1
    w2_ref[...] = w2
    i1_ref[...] = i1
    i2_ref[...] = i2


def _router(x, gate_w):
    return pl.pallas_call(
        _router_body,
        grid=(T // _RT,),
        in_specs=[
            pl.BlockSpec((_RT, D), lambda i: (i, 0)),
            pl.BlockSpec((E, D), lambda i: (0, 0)),
        ],
        out_specs=[
            pl.BlockSpec((_RT,), lambda i: (i,)),
            pl.BlockSpec((_RT,), lambda i: (i,)),
            pl.BlockSpec((_RT,), lambda i: (i,)),
            pl.BlockSpec((_RT,), lambda i: (i,)),
        ],
        out_shape=[
            jax.ShapeDtypeStruct((T,), jnp.float32),
            jax.ShapeDtypeStruct((T,), jnp.float32),
            jax.ShapeDtypeStruct((T,), jnp.int32),
            jax.ShapeDtypeStruct((T,), jnp.int32),
        ],
    )(x, gate_w)


def _dispatch_body(i1_hbm, i2_hbm,
                   rowtok_hbm, dest_hbm, be_hbm, bv_hbm, xb_hbm,
                   idx_v, cnt2d, base2d, off_v, rowtok_v, dest_v,
                   be_v, bv_v, xb_v):
    cid = lax.axis_index("c")
    sid = lax.axis_index("s")

    @pl.when(jnp.logical_and(cid == 0, sid == 0))
    def _():
        lane = lax.iota(jnp.int32, L)
        zeros = jnp.zeros((L,), jnp.int32)
        lane_e = lane * E

        pltpu.sync_copy(i1_hbm, idx_v.at[pl.ds(0, T)])
        pltpu.sync_copy(i2_hbm, idx_v.at[pl.ds(T, T)])

        for c in range(L):
            cnt2d[pl.ds(c * E, E)] = zeros

        def p1(i, _):
            e = idx_v[pl.ds(i * L, L)]
            cur = plsc.load_gather(cnt2d, [lane_e + e])
            plsc.store_scatter(cnt2d, [lane_e + e], cur + 1)
            return 0
        lax.fori_loop(0, A // L, p1, 0)

        tot = cnt2d[pl.ds(0, E)]
        for c in range(1, L):
            tot = tot + cnt2d[pl.ds(c * E, E)]

        padded = ((tot + (BLK - 1)) >> _BSH) << _BSH
        nblk = (tot + (BLK - 1)) >> _BSH
        off = plsc.cumsum(padded) - padded
        blkoff = plsc.cumsum(nblk) - nblk
        total_blocks = jnp.sum(nblk)
        off_v[...] = blkoff

        for j in range(NBLK // L):
            b = lane + j * L
            acc = jnp.zeros((L,), jnp.int32)
            for e in range(E):
                oe = plsc.load_gather(off_v, [jnp.full((L,), e, jnp.int32)])
                acc = acc + (b >= oe).astype(jnp.int32)
            be_v[pl.ds(j * L, L)] = acc - 1
            bv_v[pl.ds(j * L, L)] = (b < total_blocks).astype(jnp.int32)
            xb_v[pl.ds(j * L, L)] = jnp.minimum(b, total_blocks - 1)

        acc_v = off
        for c in range(L):
            base2d[pl.ds(c * E, E)] = acc_v
            acc_v = acc_v + cnt2d[pl.ds(c * E, E)]

        def ms(j, _):
            rowtok_v[pl.ds(j * L, L)] = (j * L + lane) & (T - 1)
            return 0
        lax.fori_loop(0, P // L, ms, 0)

        def p2(i, _):
            e = idx_v[pl.ds(i * L, L)]
            cur = plsc.load_gather(base2d, [lane_e + e])
            plsc.store_scatter(base2d, [lane_e + e], cur + 1)
            a = i * L + lane
            tok = a & (T - 1)
            kflag = a >> 12
            plsc.store_scatter(dest_v, [(tok << 1) | kflag], cur)
            plsc.store_scatter(rowtok_v, [cur], tok)
            return 0
        lax.fori_loop(0, A // L, p2, 0)

        pltpu.sync_copy(rowtok_v, rowtok_hbm)
        pltpu.sync_copy(dest_v, dest_hbm)
        pltpu.sync_copy(be_v, be_hbm)
        pltpu.sync_copy(bv_v, bv_hbm)
        pltpu.sync_copy(xb_v, xb_hbm)


def _dispatch():
  return pl.kernel(
    _dispatch_body, mesh=_mesh(), compiler_params=_SC_PARAMS,
    out_type=[
        jax.ShapeDtypeStruct((P,), jnp.int32),
        jax.ShapeDtypeStruct((A,), jnp.int32),
        jax.ShapeDtypeStruct((NBLK,), jnp.int32),
        jax.ShapeDtypeStruct((NBLK,), jnp.int32),
        jax.ShapeDtypeStruct((NBLK,), jnp.int32),
    ],
    scratch_types=[
        pltpu.VMEM((A,), jnp.int32),
        pltpu.VMEM((L * E,), jnp.int32),
        pltpu.VMEM((L * E,), jnp.int32),
        pltpu.VMEM((E,), jnp.int32),
        pltpu.VMEM((P,), jnp.int32),
        pltpu.VMEM((A,), jnp.int32),
        pltpu.VMEM((NBLK,), jnp.int32),
        pltpu.VMEM((NBLK,), jnp.int32),
        pltpu.VMEM((NBLK,), jnp.int32),
    ],
)


_GC = 8
_GN = P // NW // _GC
_GB = 4


def _gather_body(x_hbm, rowtok_hbm, xg_hbm, idx_v, *rest):
    bufs = rest[:_GB]
    gsems = rest[_GB:2 * _GB]
    ssems = rest[2 * _GB:3 * _GB]
    wid = lax.axis_index("c") * NS + lax.axis_index("s")
    base = wid * (P // NW)
    pltpu.sync_copy(rowtok_hbm.at[pl.ds(base, P // NW)], idx_v)

    def gcpy(j, b):
        return pltpu.make_async_copy(
            x_hbm.at[idx_v.at[pl.ds(j * _GC, _GC)]], bufs[b], gsems[b])

    def scpy(j, b):
        return pltpu.make_async_copy(
            bufs[b], xg_hbm.at[pl.ds(base + j * _GC, _GC)], ssems[b])

    for b in range(_GB):
        gcpy(b, b).start()

    @pl.loop(0, _GN, step=_GB)
    def _(g):
        for b in range(_GB):
            j = g + b
            gcpy(j, b).wait()
            scpy(j, b).start()
            scpy(j, b).wait()

            @pl.when(j + _GB < _GN)
            def _():
                gcpy(j + _GB, b).start()


def _gather():
  return pl.kernel(
    _gather_body, mesh=_mesh(), compiler_params=_SC_PARAMS,
    out_type=jax.ShapeDtypeStruct((P, D), jnp.float32),
    scratch_types=(
        [pltpu.VMEM((P // NW,), jnp.int32)]
        + [pltpu.VMEM((_GC, D), jnp.float32)] * _GB
        + [pltpu.SemaphoreType.DMA] * (2 * _GB)
    ),
)


def _expert_body(be_ref, bv_ref, xb_ref, xg_ref, w1_ref, w3_ref, w2_ref,
                 yg_ref):
    i = pl.program_id(0)

    @pl.when(bv_ref[i] == 1)
    def _():
        x = xg_ref[...].astype(jnp.float32)
        h1 = lax.dot_general(x, w1_ref[0], (((1,), (1,)), ((), ())),
                             preferred_element_type=jnp.float32)
        h3 = lax.dot_general(x, w3_ref[0], (((1,), (1,)), ((), ())),
                             preferred_element_type=jnp.float32)
        h = h1 * lax.logistic(h1) * h3
        yg_ref[...] = lax.dot_general(h, w2_ref[0], (((1,), (1,)), ((), ())),
                                      preferred_element_type=jnp.float32)


def _experts(xg, w1, w3, w2, be, bv, xb):
    grid_spec = pltpu.PrefetchScalarGridSpec(
        num_scalar_prefetch=3,
        grid=(NBLK,),
        in_specs=[
            pl.BlockSpec((BLK, D), lambda i, be, bv, xb: (xb[i], 0)),
            pl.BlockSpec((1, F, D), lambda i, be, bv, xb: (be[i], 0, 0)),
            pl.BlockSpec((1, F, D), lambda i, be, bv, xb: (be[i], 0, 0)),
            pl.BlockSpec((1, D, F), lambda i, be, bv, xb: (be[i], 0, 0)),
        ],
        out_specs=pl.BlockSpec((BLK, D), lambda i, be, bv, xb: (xb[i], 0)),
    )
    return pl.pallas_call(
        _expert_body,
        grid_spec=grid_spec,
        out_shape=jax.ShapeDtypeStruct((P, D), jnp.float32),
    )(be, bv, xb, xg, w1, w3, w2)


_CT = 8
_CN = T // NW // _CT
_CB = 2


def _combine_body(yg_hbm, dest_hbm, wa_hbm, wb_hbm, o_hbm,
                  dv, wv0, wv1, buf0, buf1, obuf0, obuf1,
                  g0, g1, s0, s1):
    bufs, gsems = (buf0, buf1), (g0, g1)
    obufs, ssems = (obuf0, obuf1), (s0, s1)
    wid = lax.axis_index("c") * NS + lax.axis_index("s")
    tt = T // NW
    base = wid * tt
    pltpu.sync_copy(dest_hbm.at[pl.ds(2 * base, 2 * tt)], dv)
    pltpu.sync_copy(wa_hbm.at[pl.ds(base, tt)], wv0)
    pltpu.sync_copy(wb_hbm.at[pl.ds(base, tt)], wv1)

    def gcpy(j, b):
        return pltpu.make_async_copy(
            yg_hbm.at[dv.at[pl.ds(j * 2 * _CT, 2 * _CT)]], bufs[b], gsems[b])

    def scpy(j):
        return pltpu.make_async_copy(
            obufs[j & 1], o_hbm.at[pl.ds(base + j * _CT, _CT)], ssems[j & 1])

    for b in range(_CB):
        gcpy(b, b).start()

    for j in range(_CN):
        b = j % _CB
        gcpy(j, b).wait()
        if j >= 2:
            scpy(j - 2).wait()
        for u in range(_CT):
            iu = jnp.full((L,), j * _CT + u, jnp.int32)
            wa = plsc.load_gather(wv0, [iu])
            wb = plsc.load_gather(wv1, [iu])

            @plsc.parallel_loop(0, D // L, unroll=4)
            def _(v, u=u, wa=wa, wb=wb, b=b):
                av = bufs[b][2 * u, pl.ds(v * L, L)]
                bv = bufs[b][2 * u + 1, pl.ds(v * L, L)]
                obufs[j & 1][u, pl.ds(v * L, L)] = wa * av + wb * bv
        scpy(j).start()
        if j + _CB < _CN:
            gcpy(j + _CB, b).start()
    scpy(_CN - 2).wait()
    scpy(_CN - 1).wait()


def _combine():
  return pl.kernel(
    _combine_body, mesh=_mesh(), compiler_params=_SC_PARAMS,
    out_type=jax.ShapeDtypeStruct((T, D), jnp.float32),
    scratch_types=[
        pltpu.VMEM((2 * T // NW,), jnp.int32),
        pltpu.VMEM((T // NW,), jnp.float32),
        pltpu.VMEM((T // NW,), jnp.float32),
        pltpu.VMEM((2 * _CT, D), jnp.float32),
        pltpu.VMEM((2 * _CT, D), jnp.float32),
        pltpu.VMEM((_CT, D), jnp.float32),
        pltpu.VMEM((_CT, D), jnp.float32),
        pltpu.SemaphoreType.DMA,
        pltpu.SemaphoreType.DMA,
        pltpu.SemaphoreType.DMA,
        pltpu.SemaphoreType.DMA,
    ],
)


def kernel(hidden_states, gate_w, w1, w3, w2):
    orig_shape = hidden_states.shape
    x = hidden_states.reshape(-1, D)
    wa, wb, i1, i2 = _router(x, gate_w)
    rowtok, dest, be, bv, xb = _dispatch()(i1, i2)
    xg = _gather()(x, rowtok)
    yg = _experts(xg, w1, w3, w2, be, bv, xb)
    out = _combine()(yg, dest, wa, wb)
    return out.reshape(orig_shape)


# speedup vs baseline: 1.0099x; 1.0022x over previous
import jax
import jax.numpy as jnp
from jax import lax
from jax.experimental import pallas as pl
from jax.experimental.pallas import tpu as pltpu
from jax.experimental.pallas import tpu_sc as plsc

E = 16
K = 2
D = 2048
F = 1024
T = 4096
A = T * K
BLK = 256
_BSH = BLK.bit_length() - 1
NBLK = A // BLK + E
P = NBLK * BLK
NC, NS, L = 2, 16, 16
NW = NC * NS

_SC_PARAMS = pltpu.CompilerParams(needs_layout_passes=False)


def _mesh():
    return plsc.VectorSubcoreMesh(core_axis_name="c", subcore_axis_name="s",
                                  num_cores=NC, num_subcores=NS)


_RT = 512


def _router_body(x_ref, gw_ref, w1_ref, w2_ref, i1_ref, i2_ref):
    x = x_ref[...]
    gw = gw_ref[...]
    logits = lax.dot_general(x, gw, (((1,), (1,)), ((), ())),
                             preferred_element_type=jnp.float32)
    m = jnp.max(logits, axis=-1, keepdims=True)
    ex = jnp.exp(logits - m)
    probs = ex / jnp.sum(ex, axis=-1, keepdims=True)
    i1 = jnp.argmax(probs, axis=-1).astype(jnp.int32)
    w1 = jnp.max(probs, axis=-1)
    cols = lax.broadcasted_iota(jnp.int32, probs.shape, 1)
    probs2 = jnp.where(cols == i1[:, None], -jnp.inf, probs)
    i2 = jnp.argmax(probs2, axis=-1).astype(jnp.int32)
    w2 = jnp.max(probs2, axis=-1)
    w1_ref[...] = w1
    w2_ref[...] = w2
    i1_ref[...] = i1
    i2_ref[...] = i2


def _router(x, gate_w):
    return pl.pallas_call(
        _router_body,
        grid=(T // _RT,),
        in_specs=[
            pl.BlockSpec((_RT, D), lambda i: (i, 0)),
            pl.BlockSpec((E, D), lambda i: (0, 0)),
        ],
        out_specs=[
            pl.BlockSpec((_RT,), lambda i: (i,)),
            pl.BlockSpec((_RT,), lambda i: (i,)),
            pl.BlockSpec((_RT,), lambda i: (i,)),
            pl.BlockSpec((_RT,), lambda i: (i,)),
        ],
        out_shape=[
            jax.ShapeDtypeStruct((T,), jnp.float32),
            jax.ShapeDtypeStruct((T,), jnp.float32),
            jax.ShapeDtypeStruct((T,), jnp.int32),
            jax.ShapeDtypeStruct((T,), jnp.int32),
        ],
    )(x, gate_w)


def _dispatch_body(i1_hbm, i2_hbm,
                   rowtok_hbm, dest_hbm, be_hbm, bv_hbm, xb_hbm,
                   idx_v, cnt2d, base2d, off_v, rowtok_v, dest_v,
                   be_v, bv_v, xb_v):
    cid = lax.axis_index("c")
    sid = lax.axis_index("s")

    @pl.when(jnp.logical_and(cid == 0, sid == 0))
    def _():
        lane = lax.iota(jnp.int32, L)
        zeros = jnp.zeros((L,), jnp.int32)
        lane_e = lane * E

        pltpu.sync_copy(i1_hbm, idx_v.at[pl.ds(0, T)])
        pltpu.sync_copy(i2_hbm, idx_v.at[pl.ds(T, T)])

        for c in range(L):
            cnt2d[pl.ds(c * E, E)] = zeros

        def p1(i, _):
            e = idx_v[pl.ds(i * L, L)]
            cur = plsc.load_gather(cnt2d, [lane_e + e])
            plsc.store_scatter(cnt2d, [lane_e + e], cur + 1)
            return 0
        lax.fori_loop(0, A // L, p1, 0)

        tot = cnt2d[pl.ds(0, E)]
        for c in range(1, L):
            tot = tot + cnt2d[pl.ds(c * E, E)]

        padded = ((tot + (BLK - 1)) >> _BSH) << _BSH
        nblk = (tot + (BLK - 1)) >> _BSH
        off = plsc.cumsum(padded) - padded
        blkoff = plsc.cumsum(nblk) - nblk
        total_blocks = jnp.sum(nblk)
        off_v[...] = blkoff

        for j in range(NBLK // L):
            b = lane + j * L
            acc = jnp.zeros((L,), jnp.int32)
            for e in range(E):
                oe = plsc.load_gather(off_v, [jnp.full((L,), e, jnp.int32)])
                acc = acc + (b >= oe).astype(jnp.int32)
            be_v[pl.ds(j * L, L)] = acc - 1
            bv_v[pl.ds(j * L, L)] = (b < total_blocks).astype(jnp.int32)
            xb_v[pl.ds(j * L, L)] = jnp.minimum(b, total_blocks - 1)

        acc_v = off
        for c in range(L):
            base2d[pl.ds(c * E, E)] = acc_v
            acc_v = acc_v + cnt2d[pl.ds(c * E, E)]

        def ms(j, _):
            rowtok_v[pl.ds(j * L, L)] = (j * L + lane) & (T - 1)
            return 0
        lax.fori_loop(0, P // L, ms, 0)

        def p2(i, _):
            e = idx_v[pl.ds(i * L, L)]
            cur = plsc.load_gather(base2d, [lane_e + e])
            plsc.store_scatter(base2d, [lane_e + e], cur + 1)
            a = i * L + lane
            tok = a & (T - 1)
            kflag = a >> 12
            plsc.store_scatter(dest_v, [(tok << 1) | kflag], cur)
            plsc.store_scatter(rowtok_v, [cur], tok)
            return 0
        lax.fori_loop(0, A // L, p2, 0)

        pltpu.sync_copy(rowtok_v, rowtok_hbm)
        pltpu.sync_copy(dest_v, dest_hbm)
        pltpu.sync_copy(be_v, be_hbm)
        pltpu.sync_copy(bv_v, bv_hbm)
        pltpu.sync_copy(xb_v, xb_hbm)


def _dispatch():
  return pl.kernel(
    _dispatch_body, mesh=_mesh(), compiler_params=_SC_PARAMS,
    out_type=[
        jax.ShapeDtypeStruct((P,), jnp.int32),
        jax.ShapeDtypeStruct((A,), jnp.int32),
        jax.ShapeDtypeStruct((NBLK,), jnp.int32),
        jax.ShapeDtypeStruct((NBLK,), jnp.int32),
        jax.ShapeDtypeStruct((NBLK,), jnp.int32),
    ],
    scratch_types=[
        pltpu.VMEM((A,), jnp.int32),
        pltpu.VMEM((L * E,), jnp.int32),
        pltpu.VMEM((L * E,), jnp.int32),
        pltpu.VMEM((E,), jnp.int32),
        pltpu.VMEM((P,), jnp.int32),
        pltpu.VMEM((A,), jnp.int32),
        pltpu.VMEM((NBLK,), jnp.int32),
        pltpu.VMEM((NBLK,), jnp.int32),
        pltpu.VMEM((NBLK,), jnp.int32),
    ],
)


_GC = 8
_GN = P // NW // _GC
_GB = 4


def _gather_body(x_hbm, rowtok_hbm, xg_hbm, idx_v, *rest):
    bufs = rest[:_GB]
    gsems = rest[_GB:2 * _GB]
    ssems = rest[2 * _GB:3 * _GB]
    wid = lax.axis_index("c") * NS + lax.axis_index("s")
    base = wid * (P // NW)
    pltpu.sync_copy(rowtok_hbm.at[pl.ds(base, P // NW)], idx_v)

    def gcpy(j, b):
        return pltpu.make_async_copy(
            x_hbm.at[idx_v.at[pl.ds(j * _GC, _GC)]], bufs[b], gsems[b])

    def scpy(j, b):
        return pltpu.make_async_copy(
            bufs[b], xg_hbm.at[pl.ds(base + j * _GC, _GC)], ssems[b])

    for b in range(_GB):
        gcpy(b, b).start()

    @pl.loop(0, _GN, step=_GB)
    def _(g):
        for b in range(_GB):
            j = g + b
            gcpy(j, b).wait()
            scpy(j, b).start()
            scpy(j, b).wait()

            @pl.when(j + _GB < _GN)
            def _():
                gcpy(j + _GB, b).start()


def _gather():
  return pl.kernel(
    _gather_body, mesh=_mesh(), compiler_params=_SC_PARAMS,
    out_type=jax.ShapeDtypeStruct((P, D), jnp.float32),
    scratch_types=(
        [pltpu.VMEM((P // NW,), jnp.int32)]
        + [pltpu.VMEM((_GC, D), jnp.float32)] * _GB
        + [pltpu.SemaphoreType.DMA] * (2 * _GB)
    ),
)


def _expert_body(be_ref, bv_ref, xb_ref, xg_ref, w1_ref, w3_ref, w2_ref,
                 yg_ref):
    i = pl.program_id(0)

    @pl.when(bv_ref[i] == 1)
    def _():
        x = xg_ref[...].astype(jnp.float32)
        h1 = lax.dot_general(x, w1_ref[0], (((1,), (1,)), ((), ())),
                             preferred_element_type=jnp.float32)
        h3 = lax.dot_general(x, w3_ref[0], (((1,), (1,)), ((), ())),
                             preferred_element_type=jnp.float32)
        h = h1 * lax.logistic(h1) * h3
        yg_ref[...] = lax.dot_general(h, w2_ref[0], (((1,), (1,)), ((), ())),
                                      preferred_element_type=jnp.float32)


def _experts(xg, w1, w3, w2, be, bv, xb):
    grid_spec = pltpu.PrefetchScalarGridSpec(
        num_scalar_prefetch=3,
        grid=(NBLK,),
        in_specs=[
            pl.BlockSpec((BLK, D), lambda i, be, bv, xb: (xb[i], 0)),
            pl.BlockSpec((1, F, D), lambda i, be, bv, xb: (be[i], 0, 0)),
            pl.BlockSpec((1, F, D), lambda i, be, bv, xb: (be[i], 0, 0)),
            pl.BlockSpec((1, D, F), lambda i, be, bv, xb: (be[i], 0, 0)),
        ],
        out_specs=pl.BlockSpec((BLK, D), lambda i, be, bv, xb: (xb[i], 0)),
    )
    return pl.pallas_call(
        _expert_body,
        grid_spec=grid_spec,
        out_shape=jax.ShapeDtypeStruct((P, D), jnp.float32),
    )(be, bv, xb, xg, w1, w3, w2)


_CT = 8
_CN = T // NW // _CT
_CB = 2


def _combine_body(yg_hbm, dest_hbm, wa_hbm, wb_hbm, o_hbm,
                  dv, wv0, wv1, buf0, buf1, obuf0, obuf1,
                  g0, g1, s0, s1):
    bufs, gsems = (buf0, buf1), (g0, g1)
    obufs, ssems = (obuf0, obuf1), (s0, s1)
    wid = lax.axis_index("c") * NS + lax.axis_index("s")
    tt = T // NW
    base = wid * tt
    pltpu.sync_copy(dest_hbm.at[pl.ds(2 * base, 2 * tt)], dv)
    pltpu.sync_copy(wa_hbm.at[pl.ds(base, tt)], wv0)
    pltpu.sync_copy(wb_hbm.at[pl.ds(base, tt)], wv1)

    def gcpy(j, b):
        return pltpu.make_async_copy(
            yg_hbm.at[dv.at[pl.ds(j * 2 * _CT, 2 * _CT)]], bufs[b], gsems[b])

    def scpy(j):
        return pltpu.make_async_copy(
            obufs[j & 1], o_hbm.at[pl.ds(base + j * _CT, _CT)], ssems[j & 1])

    for b in range(_CB):
        gcpy(b, b).start()

    for j in range(_CN):
        b = j % _CB
        gcpy(j, b).wait()
        if j >= 2:
            scpy(j - 2).wait()
        for u in range(_CT):
            iu = jnp.full((L,), j * _CT + u, jnp.int32)
            wa = plsc.load_gather(wv0, [iu])
            wb = plsc.load_gather(wv1, [iu])

            @plsc.parallel_loop(0, D // L, unroll=4)
            def _(v, u=u, wa=wa, wb=wb, b=b):
                av = bufs[b][2 * u, pl.ds(v * L, L)]
                bv = bufs[b][2 * u + 1, pl.ds(v * L, L)]
                obufs[j & 1][u, pl.ds(v * L, L)] = wa * av + wb * bv
        scpy(j).start()
        if j + _CB < _CN:
            gcpy(j + _CB, b).start()
    scpy(_CN - 2).wait()
    scpy(_CN - 1).wait()


def _combine():
  return pl.kernel(
    _combine_body, mesh=_mesh(), compiler_params=_SC_PARAMS,
    out_type=jax.ShapeDtypeStruct((T, D), jnp.float32),
    scratch_types=[
        pltpu.VMEM((2 * T // NW,), jnp.int32),
        pltpu.VMEM((T // NW,), jnp.float32),
        pltpu.VMEM((T // NW,), jnp.float32),
        pltpu.VMEM((2 * _CT, D), jnp.float32),
        pltpu.VMEM((2 * _CT, D), jnp.float32),
        pltpu.VMEM((_CT, D), jnp.float32),
        pltpu.VMEM((_CT, D), jnp.float32),
        pltpu.SemaphoreType.DMA,
        pltpu.SemaphoreType.DMA,
        pltpu.SemaphoreType.DMA,
        pltpu.SemaphoreType.DMA,
    ],
)


def kernel(hidden_states, gate_w, w1, w3, w2):
    orig_shape = hidden_states.shape
    x = hidden_states.reshape(-1, D)
    wa, wb, i1, i2 = _router(x, gate_w)
    rowtok, dest, be, bv, xb = _dispatch()(i1, i2)
    xg = _gather()(x, rowtok)
    yg = _experts(xg, w1, w3, w2, be, bv, xb)
    out = _combine()(yg, dest, wa, wb)
    return out.reshape(orig_shape)


# speedup vs baseline: 1.0205x; 1.0105x over previous
import jax
import jax.numpy as jnp
from jax import lax
from jax.experimental import pallas as pl
from jax.experimental.pallas import tpu as pltpu
from jax.experimental.pallas import tpu_sc as plsc

E = 16
K = 2
D = 2048
F = 1024
T = 4096
A = T * K
BLK = 256
_BSH = BLK.bit_length() - 1
NBLK = A // BLK + E
P = NBLK * BLK
NC, NS, L = 2, 16, 16
NW = NC * NS
_HE = L * E

_SC_PARAMS = pltpu.CompilerParams(needs_layout_passes=False)


def _mesh():
    return plsc.VectorSubcoreMesh(core_axis_name="c", subcore_axis_name="s",
                                  num_cores=NC, num_subcores=NS)


_RT = 512


def _router_body(x_ref, gw_ref, w1_ref, w2_ref, i1_ref, i2_ref):
    x = x_ref[...]
    gw = gw_ref[...]
    logits = lax.dot_general(x, gw, (((1,), (1,)), ((), ())),
                             preferred_element_type=jnp.float32)
    m = jnp.max(logits, axis=-1, keepdims=True)
    ex = jnp.exp(logits - m)
    probs = ex / jnp.sum(ex, axis=-1, keepdims=True)
    i1 = jnp.argmax(probs, axis=-1).astype(jnp.int32)
    w1 = jnp.max(probs, axis=-1)
    cols = lax.broadcasted_iota(jnp.int32, probs.shape, 1)
    probs2 = jnp.where(cols == i1[:, None], -jnp.inf, probs)
    i2 = jnp.argmax(probs2, axis=-1).astype(jnp.int32)
    w2 = jnp.max(probs2, axis=-1)
    w1_ref[...] = w1
    w2_ref[...] = w2
    i1_ref[...] = i1
    i2_ref[...] = i2


def _router(x, gate_w):
    return pl.pallas_call(
        _router_body,
        grid=(T // _RT,),
        in_specs=[
            pl.BlockSpec((_RT, D), lambda i: (i, 0)),
            pl.BlockSpec((E, D), lambda i: (0, 0)),
        ],
        out_specs=[
            pl.BlockSpec((_RT,), lambda i: (i,)),
            pl.BlockSpec((_RT,), lambda i: (i,)),
            pl.BlockSpec((_RT,), lambda i: (i,)),
            pl.BlockSpec((_RT,), lambda i: (i,)),
        ],
        out_shape=[
            jax.ShapeDtypeStruct((T,), jnp.float32),
            jax.ShapeDtypeStruct((T,), jnp.float32),
            jax.ShapeDtypeStruct((T,), jnp.int32),
            jax.ShapeDtypeStruct((T,), jnp.int32),
        ],
    )(x, gate_w)


def _dispatch_body(i1_hbm, i2_hbm,
                   rowtok_hbm, dest_hbm, be_hbm, bv_hbm, xb_hbm,
                   idx_v, cnt2d, base2d, off_v, rowtok_v, dest_v,
                   be_v, bv_v, xb_v):
    cid = lax.axis_index("c")
    sid = lax.axis_index("s")

    @pl.when(jnp.logical_and(cid == 0, sid == 0))
    def _():
        lane = lax.iota(jnp.int32, L)
        zeros = jnp.zeros((L,), jnp.int32)
        lane_e = lane * E

        pltpu.sync_copy(i1_hbm, idx_v.at[pl.ds(0, T)])
        pltpu.sync_copy(i2_hbm, idx_v.at[pl.ds(T, T)])

        for c in range(2 * L):
            cnt2d[pl.ds(c * E, E)] = zeros

        def p1(i, _):
            ea = idx_v[pl.ds(2 * i * L, L)]
            eb = idx_v[pl.ds((2 * i + 1) * L, L)]
            ca = plsc.load_gather(cnt2d, [lane_e + ea])
            cb = plsc.load_gather(cnt2d, [_HE + lane_e + eb])
            plsc.store_scatter(cnt2d, [lane_e + ea], ca + 1)
            plsc.store_scatter(cnt2d, [_HE + lane_e + eb], cb + 1)
            return 0
        lax.fori_loop(0, A // (2 * L), p1, 0)

        tot = cnt2d[pl.ds(0, E)]
        for c in range(1, 2 * L):
            tot = tot + cnt2d[pl.ds(c * E, E)]

        padded = ((tot + (BLK - 1)) >> _BSH) << _BSH
        nblk = (tot + (BLK - 1)) >> _BSH
        off = plsc.cumsum(padded) - padded
        blkoff = plsc.cumsum(nblk) - nblk
        total_blocks = jnp.sum(nblk)
        off_v[...] = blkoff

        for j in range(NBLK // L):
            b = lane + j * L
            acc = jnp.zeros((L,), jnp.int32)
            for e in range(E):
                oe = plsc.load_gather(off_v, [jnp.full((L,), e, jnp.int32)])
                acc = acc + (b >= oe).astype(jnp.int32)
            be_v[pl.ds(j * L, L)] = acc - 1
            bv_v[pl.ds(j * L, L)] = (b < total_blocks).astype(jnp.int32)
            xb_v[pl.ds(j * L, L)] = jnp.minimum(b, total_blocks - 1)

        acc_v = off
        for c in range(2 * L):
            base2d[pl.ds(c * E, E)] = acc_v
            acc_v = acc_v + cnt2d[pl.ds(c * E, E)]

        def ms(j, _):
            rowtok_v[pl.ds(j * L, L)] = (j * L + lane) & (T - 1)
            return 0
        lax.fori_loop(0, P // L, ms, 0)

        def p2(i, _):
            ea = idx_v[pl.ds(2 * i * L, L)]
            eb = idx_v[pl.ds((2 * i + 1) * L, L)]
            ca = plsc.load_gather(base2d, [lane_e + ea])
            cb = plsc.load_gather(base2d, [_HE + lane_e + eb])
            plsc.store_scatter(base2d, [lane_e + ea], ca + 1)
            plsc.store_scatter(base2d, [_HE + lane_e + eb], cb + 1)
            aa = 2 * i * L + lane
            ab = aa + L
            ta = aa & (T - 1)
            tb = ab & (T - 1)
            plsc.store_scatter(dest_v, [(ta << 1) | (aa >> 12)], ca)
            plsc.store_scatter(dest_v, [(tb << 1) | (ab >> 12)], cb)
            plsc.store_scatter(rowtok_v, [ca], ta)
            plsc.store_scatter(rowtok_v, [cb], tb)
            return 0
        lax.fori_loop(0, A // (2 * L), p2, 0)

        pltpu.sync_copy(rowtok_v, rowtok_hbm)
        pltpu.sync_copy(dest_v, dest_hbm)
        pltpu.sync_copy(be_v, be_hbm)
        pltpu.sync_copy(bv_v, bv_hbm)
        pltpu.sync_copy(xb_v, xb_hbm)


def _dispatch():
  return pl.kernel(
    _dispatch_body, mesh=_mesh(), compiler_params=_SC_PARAMS,
    out_type=[
        jax.ShapeDtypeStruct((P,), jnp.int32),
        jax.ShapeDtypeStruct((A,), jnp.int32),
        jax.ShapeDtypeStruct((NBLK,), jnp.int32),
        jax.ShapeDtypeStruct((NBLK,), jnp.int32),
        jax.ShapeDtypeStruct((NBLK,), jnp.int32),
    ],
    scratch_types=[
        pltpu.VMEM((A,), jnp.int32),
        pltpu.VMEM((2 * L * E,), jnp.int32),
        pltpu.VMEM((2 * L * E,), jnp.int32),
        pltpu.VMEM((E,), jnp.int32),
        pltpu.VMEM((P,), jnp.int32),
        pltpu.VMEM((A,), jnp.int32),
        pltpu.VMEM((NBLK,), jnp.int32),
        pltpu.VMEM((NBLK,), jnp.int32),
        pltpu.VMEM((NBLK,), jnp.int32),
    ],
)


_GC = 8
_GN = P // NW // _GC
_GB = 4


def _gather_body(x_hbm, rowtok_hbm, xg_hbm, idx_v, *rest):
    bufs = rest[:_GB]
    gsems = rest[_GB:2 * _GB]
    ssems = rest[2 * _GB:3 * _GB]
    wid = lax.axis_index("c") * NS + lax.axis_index("s")
    base = wid * (P // NW)
    pltpu.sync_copy(rowtok_hbm.at[pl.ds(base, P // NW)], idx_v)

    def gcpy(j, b):
        return pltpu.make_async_copy(
            x_hbm.at[idx_v.at[pl.ds(j * _GC, _GC)]], bufs[b], gsems[b])

    def scpy(j, b):
        return pltpu.make_async_copy(
            bufs[b], xg_hbm.at[pl.ds(base + j * _GC, _GC)], ssems[b])

    for b in range(_GB):
        gcpy(b, b).start()

    @pl.loop(0, _GN, step=_GB)
    def _(g):
        for b in range(_GB):
            j = g + b
            gcpy(j, b).wait()
            scpy(j, b).start()
            scpy(j, b).wait()

            @pl.when(j + _GB < _GN)
            def _():
                gcpy(j + _GB, b).start()


def _gather():
  return pl.kernel(
    _gather_body, mesh=_mesh(), compiler_params=_SC_PARAMS,
    out_type=jax.ShapeDtypeStruct((P, D), jnp.float32),
    scratch_types=(
        [pltpu.VMEM((P // NW,), jnp.int32)]
        + [pltpu.VMEM((_GC, D), jnp.float32)] * _GB
        + [pltpu.SemaphoreType.DMA] * (2 * _GB)
    ),
)


def _expert_body(be_ref, bv_ref, xb_ref, xg_ref, w1_ref, w3_ref, w2_ref,
                 yg_ref):
    i = pl.program_id(0)

    @pl.when(bv_ref[i] == 1)
    def _():
        x = xg_ref[...].astype(jnp.float32)
        h1 = lax.dot_general(x, w1_ref[0], (((1,), (1,)), ((), ())),
                             preferred_element_type=jnp.float32)
        h3 = lax.dot_general(x, w3_ref[0], (((1,), (1,)), ((), ())),
                             preferred_element_type=jnp.float32)
        h = h1 * lax.logistic(h1) * h3
        yg_ref[...] = lax.dot_general(h, w2_ref[0], (((1,), (1,)), ((), ())),
                                      preferred_element_type=jnp.float32)


def _experts(xg, w1, w3, w2, be, bv, xb):
    grid_spec = pltpu.PrefetchScalarGridSpec(
        num_scalar_prefetch=3,
        grid=(NBLK,),
        in_specs=[
            pl.BlockSpec((BLK, D), lambda i, be, bv, xb: (xb[i], 0)),
            pl.BlockSpec((1, F, D), lambda i, be, bv, xb: (be[i], 0, 0)),
            pl.BlockSpec((1, F, D), lambda i, be, bv, xb: (be[i], 0, 0)),
            pl.BlockSpec((1, D, F), lambda i, be, bv, xb: (be[i], 0, 0)),
        ],
        out_specs=pl.BlockSpec((BLK, D), lambda i, be, bv, xb: (xb[i], 0)),
    )
    return pl.pallas_call(
        _expert_body,
        grid_spec=grid_spec,
        out_shape=jax.ShapeDtypeStruct((P, D), jnp.float32),
    )(be, bv, xb, xg, w1, w3, w2)


_CT = 8
_CN = T // NW // _CT
_CB = 2


def _combine_body(yg_hbm, dest_hbm, wa_hbm, wb_hbm, o_hbm,
                  dv, wv0, wv1, buf0, buf1, obuf0, obuf1,
                  g0, g1, s0, s1):
    bufs, gsems = (buf0, buf1), (g0, g1)
    obufs, ssems = (obuf0, obuf1), (s0, s1)
    wid = lax.axis_index("c") * NS + lax.axis_index("s")
    tt = T // NW
    base = wid * tt
    pltpu.sync_copy(dest_hbm.at[pl.ds(2 * base, 2 * tt)], dv)
    pltpu.sync_copy(wa_hbm.at[pl.ds(base, tt)], wv0)
    pltpu.sync_copy(wb_hbm.at[pl.ds(base, tt)], wv1)

    def gcpy(j, b):
        return pltpu.make_async_copy(
            yg_hbm.at[dv.at[pl.ds(j * 2 * _CT, 2 * _CT)]], bufs[b], gsems[b])

    def scpy(j):
        return pltpu.make_async_copy(
            obufs[j & 1], o_hbm.at[pl.ds(base + j * _CT, _CT)], ssems[j & 1])

    for b in range(_CB):
        gcpy(b, b).start()

    for j in range(_CN):
        b = j % _CB
        gcpy(j, b).wait()
        if j >= 2:
            scpy(j - 2).wait()
        for u in range(_CT):
            iu = jnp.full((L,), j * _CT + u, jnp.int32)
            wa = plsc.load_gather(wv0, [iu])
            wb = plsc.load_gather(wv1, [iu])

            @plsc.parallel_loop(0, D // L, unroll=4)
            def _(v, u=u, wa=wa, wb=wb, b=b):
                av = bufs[b][2 * u, pl.ds(v * L, L)]
                bv = bufs[b][2 * u + 1, pl.ds(v * L, L)]
                obufs[j & 1][u, pl.ds(v * L, L)] = wa * av + wb * bv
        scpy(j).start()
        if j + _CB < _CN:
            gcpy(j + _CB, b).start()
    scpy(_CN - 2).wait()
    scpy(_CN - 1).wait()


def _combine():
  return pl.kernel(
    _combine_body, mesh=_mesh(), compiler_params=_SC_PARAMS,
    out_type=jax.ShapeDtypeStruct((T, D), jnp.float32),
    scratch_types=[
        pltpu.VMEM((2 * T // NW,), jnp.int32),
        pltpu.VMEM((T // NW,), jnp.float32),
        pltpu.VMEM((T // NW,), jnp.float32),
        pltpu.VMEM((2 * _CT, D), jnp.float32),
        pltpu.VMEM((2 * _CT, D), jnp.float32),
        pltpu.VMEM((_CT, D), jnp.float32),
        pltpu.VMEM((_CT, D), jnp.float32),
        pltpu.SemaphoreType.DMA,
        pltpu.SemaphoreType.DMA,
        pltpu.SemaphoreType.DMA,
        pltpu.SemaphoreType.DMA,
    ],
)


def kernel(hidden_states, gate_w, w1, w3, w2):
    orig_shape = hidden_states.shape
    x = hidden_states.reshape(-1, D)
    wa, wb, i1, i2 = _router(x, gate_w)
    rowtok, dest, be, bv, xb = _dispatch()(i1, i2)
    xg = _gather()(x, rowtok)
    yg = _experts(xg, w1, w3, w2, be, bv, xb)
    out = _combine()(yg, dest, wa, wb)
    return out.reshape(orig_shape)
